# baseline (speedup 1.0000x reference)
import functools

import jax
import jax.numpy as jnp
from jax import lax
from jax.experimental import pallas as pl
from jax.experimental.pallas import tpu as pltpu
from jax.experimental.pallas import tpu_sc as plsc


def _emb_call(B, H, D, rows_per_w):
    n_pairs = rows_per_w // 2
    mesh = plsc.VectorSubcoreMesh(core_axis_name="c", subcore_axis_name="s")

    @functools.partial(
        pl.kernel,
        mesh=mesh,
        out_type=jax.ShapeDtypeStruct((B, H, D), jnp.float32),
        scratch_types=[
            pltpu.VMEM((rows_per_w, H), jnp.int32),
            pltpu.VMEM((H, D), jnp.float32),
            pltpu.VMEM((H, D), jnp.float32),
            pltpu.SemaphoreType.DMA,
            pltpu.SemaphoreType.DMA,
            pltpu.SemaphoreType.DMA,
            pltpu.SemaphoreType.DMA,
        ],
        compiler_params=pltpu.CompilerParams(use_tc_tiling_on_sc=False),
    )
    def _emb(idx_hbm, table_hbm, out_hbm, idx_v, rows0, rows1,
             gsem0, gsem1, osem0, osem1):
        wid = lax.axis_index("s") * 2 + lax.axis_index("c")
        row_base = wid * rows_per_w
        pltpu.sync_copy(idx_hbm.at[pl.ds(row_base, rows_per_w)], idx_v)

        def gdesc(r, rows, sem):
            return pltpu.make_async_copy(
                table_hbm.at[idx_v.at[r]], rows, sem)

        def odesc(r, rows, sem):
            return pltpu.make_async_copy(
                rows, out_hbm.at[row_base + r], sem)

        gdesc(0, rows0, gsem0).start()

        def pair(i, carry):
            r0 = 2 * i

            @pl.when(i > 0)
            def _():
                odesc(r0 - 1, rows1, osem1).wait()

            gdesc(r0 + 1, rows1, gsem1).start()
            gdesc(r0, rows0, gsem0).wait()
            odesc(r0, rows0, osem0).start()
            odesc(r0, rows0, osem0).wait()

            @pl.when(r0 + 2 < rows_per_w)
            def _():
                gdesc(r0 + 2, rows0, gsem0).start()

            gdesc(r0 + 1, rows1, gsem1).wait()
            odesc(r0 + 1, rows1, osem1).start()
            return carry

        lax.fori_loop(0, n_pairs, pair, 0)
        odesc(rows_per_w - 1, rows1, osem1).wait()

    return _emb


def kernel(input, weight):
    B, H = input.shape
    V, D = weight.shape
    NW = 32
    rows_per_w = B // NW
    return _emb_call(B, H, D, rows_per_w)(input, weight)


# speedup vs baseline: 1.1089x; 1.1089x over previous
import functools

import jax
import jax.numpy as jnp
from jax import lax
from jax.experimental import pallas as pl
from jax.experimental.pallas import tpu as pltpu
from jax.experimental.pallas import tpu_sc as plsc


def _emb_call(B, H, D, rows_per_w, CP):
    pairs_per_w = rows_per_w * H // 2
    n_chunks = pairs_per_w // CP
    n_pairs = n_chunks // 2
    mesh = plsc.VectorSubcoreMesh(core_axis_name="c", subcore_axis_name="s")

    @functools.partial(
        pl.kernel,
        mesh=mesh,
        out_type=jax.ShapeDtypeStruct((B * H // 2, 2 * D), jnp.float32),
        scratch_types=[
            pltpu.VMEM((rows_per_w, H), jnp.int32),
            pltpu.VMEM((pairs_per_w,), jnp.int32),
            pltpu.VMEM((pairs_per_w,), jnp.int32),
            pltpu.VMEM((2, CP, D), jnp.float32),
            pltpu.VMEM((2, CP, D), jnp.float32),
            pltpu.SemaphoreType.DMA,
            pltpu.SemaphoreType.DMA,
            pltpu.SemaphoreType.DMA,
            pltpu.SemaphoreType.DMA,
        ],
        compiler_params=pltpu.CompilerParams(use_tc_tiling_on_sc=False, needs_layout_passes=False),
    )
    def _emb(idx_hbm, table_hbm, out_hbm, idx_v, eidx, oidx, rows0, rows1,
             gsem0, gsem1, osem0, osem1):
        wid = lax.axis_index("s") * 2 + lax.axis_index("c")
        row_base = wid * rows_per_w
        pair_base = wid * pairs_per_w
        pltpu.sync_copy(idx_hbm.at[pl.ds(row_base, rows_per_w)], idx_v)

        lanes = lax.iota(jnp.int32, 16)

        def repack(j, carry):
            p = j * 16 + lanes
            pe = p * 2
            po = pe + 1
            ev = plsc.load_gather(idx_v, [pe // H, pe % H])
            ov = plsc.load_gather(idx_v, [po // H, po % H])
            eidx[pl.ds(j * 16, 16)] = ev
            oidx[pl.ds(j * 16, 16)] = ov
            return carry

        lax.fori_loop(0, pairs_per_w // 16, repack, 0)

        def gdesc(g, rows, sem):
            base = g * CP
            return (
                pltpu.make_async_copy(
                    table_hbm.at[eidx.at[pl.ds(base, CP)]], rows.at[0], sem),
                pltpu.make_async_copy(
                    table_hbm.at[oidx.at[pl.ds(base, CP)]], rows.at[1], sem),
            )

        def odesc(g, rows, sem):
            base = pair_base + g * CP
            return (
                pltpu.make_async_copy(
                    rows.at[0], out_hbm.at[pl.ds(base, CP), pl.ds(0, D)], sem),
                pltpu.make_async_copy(
                    rows.at[1], out_hbm.at[pl.ds(base, CP), pl.ds(D, D)], sem),
            )

        def start(descs):
            for d in descs:
                d.start()

        def wait(descs):
            for d in descs:
                d.wait()

        start(gdesc(0, rows0, gsem0))

        def pair(i, carry):
            g0 = 2 * i

            @pl.when(i > 0)
            def _():
                wait(odesc(g0 - 1, rows1, osem1))

            start(gdesc(g0 + 1, rows1, gsem1))
            wait(gdesc(g0, rows0, gsem0))
            start(odesc(g0, rows0, osem0))
            wait(odesc(g0, rows0, osem0))

            @pl.when(g0 + 2 < n_chunks)
            def _():
                start(gdesc(g0 + 2, rows0, gsem0))

            wait(gdesc(g0 + 1, rows1, gsem1))
            start(odesc(g0 + 1, rows1, osem1))
            return carry

        lax.fori_loop(0, n_pairs, pair, 0)
        wait(odesc(n_chunks - 1, rows1, osem1))

    return _emb


def kernel(input, weight):
    B, H = input.shape
    V, D = weight.shape
    NW = 32
    rows_per_w = B // NW
    CP = 256
    out = _emb_call(B, H, D, rows_per_w, CP)(input, weight)
    return out.reshape(B, H, D)


# speedup vs baseline: 1.3692x; 1.2347x over previous
import functools

import jax
import jax.numpy as jnp
from jax import lax
from jax.experimental import pallas as pl
from jax.experimental.pallas import tpu as pltpu
from jax.experimental.pallas import tpu_sc as plsc

_NBUF = 8


def _emb_call(B, H, D, rows_per_w):
    mesh = plsc.VectorSubcoreMesh(core_axis_name="c", subcore_axis_name="s")
    n_outer = rows_per_w // _NBUF

    @functools.partial(
        pl.kernel,
        mesh=mesh,
        out_type=jax.ShapeDtypeStruct((B, 56, 2 * D), jnp.float32),
        scratch_types=(
            [pltpu.VMEM((rows_per_w, H), jnp.int32)]
            + [pltpu.VMEM((H, 2 * D), jnp.float32) for _ in range(_NBUF)]
            + [pltpu.SemaphoreType.DMA for _ in range(2 * _NBUF)]
        ),
        compiler_params=pltpu.CompilerParams(
            use_tc_tiling_on_sc=False, needs_layout_passes=False),
    )
    def _emb(idx_hbm, table_hbm, out_hbm, idx_v, *bufs_and_sems):
        rows = bufs_and_sems[:_NBUF]
        gsem = bufs_and_sems[_NBUF:2 * _NBUF]
        osem = bufs_and_sems[2 * _NBUF:]
        wid = lax.axis_index("s") * 2 + lax.axis_index("c")
        row_base = wid * rows_per_w
        pltpu.sync_copy(idx_hbm.at[pl.ds(row_base, rows_per_w)], idx_v)

        def gdesc(r, b):
            return pltpu.make_async_copy(
                table_hbm.at[idx_v.at[r]], rows[b], gsem[b])

        def odesc(r, b):
            return pltpu.make_async_copy(
                rows[b], out_hbm.at[row_base + r, pl.ds(0, H)], osem[b])

        for b in range(_NBUF):
            gdesc(b, b).start()

        def step(outer, carry):
            r0 = outer * _NBUF
            for b in range(_NBUF):
                r = r0 + b
                gdesc(r, b).wait()
                odesc(r, b).start()

                @pl.when(outer < n_outer - 1)
                def _(r=r, b=b):
                    odesc(r, b).wait()
                    gdesc(r + _NBUF, b).start()

            return carry

        lax.fori_loop(0, n_outer, step, 0)
        for b in range(_NBUF):
            odesc(rows_per_w - _NBUF + b, b).wait()

    return _emb


def kernel(input, weight):
    B, H = input.shape
    V, D = weight.shape
    NW = 32
    rows_per_w = B // NW
    wpad = jnp.pad(weight, ((0, 0), (0, D)))
    out56 = _emb_call(B, H, D, rows_per_w)(input, wpad)
    return out56[:, :H, :D]


# speedup vs baseline: 1.4903x; 1.0884x over previous
import functools

import jax
import jax.numpy as jnp
from jax import lax
from jax.experimental import pallas as pl
from jax.experimental.pallas import tpu as pltpu
from jax.experimental.pallas import tpu_sc as plsc

_NBUF = 8


def _emb_call(B, H, D, rows_per_w):
    mesh = plsc.VectorSubcoreMesh(core_axis_name="c", subcore_axis_name="s")
    n_outer = rows_per_w // _NBUF

    @functools.partial(
        pl.kernel,
        mesh=mesh,
        out_type=jax.ShapeDtypeStruct((B, 56, 2 * D), jnp.float32),
        scratch_types=(
            [pltpu.VMEM((rows_per_w, H), jnp.int32)]
            + [pltpu.VMEM((H, D), jnp.float32) for _ in range(_NBUF)]
            + [pltpu.SemaphoreType.DMA for _ in range(2 * _NBUF)]
        ),
        compiler_params=pltpu.CompilerParams(
            use_tc_tiling_on_sc=False, needs_layout_passes=False),
    )
    def _emb(idx_hbm, table_hbm, out_hbm, idx_v, *bufs_and_sems):
        rows = bufs_and_sems[:_NBUF]
        gsem = bufs_and_sems[_NBUF:2 * _NBUF]
        osem = bufs_and_sems[2 * _NBUF:]
        wid = lax.axis_index("s") * 2 + lax.axis_index("c")
        row_base = wid * rows_per_w
        pltpu.sync_copy(idx_hbm.at[pl.ds(row_base, rows_per_w)], idx_v)

        def gdesc(r, b):
            return pltpu.make_async_copy(
                table_hbm.at[idx_v.at[r]], rows[b], gsem[b])

        def odesc(r, b):
            return pltpu.make_async_copy(
                rows[b], out_hbm.at[row_base + r, pl.ds(0, H), pl.ds(0, D)],
                osem[b])

        for b in range(_NBUF):
            gdesc(b, b).start()

        def step(outer, carry):
            r0 = outer * _NBUF
            for b in range(_NBUF):
                r = r0 + b
                gdesc(r, b).wait()
                odesc(r, b).start()

                @pl.when(outer < n_outer - 1)
                def _(r=r, b=b):
                    odesc(r, b).wait()
                    gdesc(r + _NBUF, b).start()

            return carry

        lax.fori_loop(0, n_outer, step, 0)
        for b in range(_NBUF):
            odesc(rows_per_w - _NBUF + b, b).wait()

    return _emb


def kernel(input, weight):
    B, H = input.shape
    V, D = weight.shape
    NW = 32
    rows_per_w = B // NW
    out56 = _emb_call(B, H, D, rows_per_w)(input, weight)
    return out56[:, :H, :D]


# speedup vs baseline: 1.8774x; 1.2598x over previous
import functools

import jax
import jax.numpy as jnp
from jax import lax
from jax.experimental import pallas as pl
from jax.experimental.pallas import tpu as pltpu
from jax.experimental.pallas import tpu_sc as plsc

_NBUF = 8


def _emb_call(B, H, D, rows_per_w):
    mesh = plsc.VectorSubcoreMesh(core_axis_name="c", subcore_axis_name="s")
    n_outer = rows_per_w // _NBUF

    @functools.partial(
        pl.kernel,
        mesh=mesh,
        out_type=jax.ShapeDtypeStruct((B, 56, 2 * D), jnp.float32),
        scratch_types=(
            [pltpu.VMEM((rows_per_w, H), jnp.int32)]
            + [pltpu.VMEM((H, 2 * D), jnp.float32) for _ in range(_NBUF)]
            + [pltpu.SemaphoreType.DMA for _ in range(2 * _NBUF)]
        ),
        compiler_params=pltpu.CompilerParams(
            use_tc_tiling_on_sc=False, needs_layout_passes=False),
    )
    def _emb(idx_hbm, table_hbm, out_hbm, idx_v, *bufs_and_sems):
        rows = bufs_and_sems[:_NBUF]
        gsem = bufs_and_sems[_NBUF:2 * _NBUF]
        osem = bufs_and_sems[2 * _NBUF:]
        wid = lax.axis_index("s") * 2 + lax.axis_index("c")
        row_base = wid * rows_per_w
        pltpu.sync_copy(idx_hbm.at[pl.ds(row_base, rows_per_w)], idx_v)

        def gdesc(r, b):
            return pltpu.make_async_copy(
                table_hbm.at[idx_v.at[r]], rows[b], gsem[b])

        def odesc(r, b):
            return pltpu.make_async_copy(
                rows[b], out_hbm.at[row_base + r, pl.ds(0, H)], osem[b])

        for b in range(_NBUF):
            gdesc(b, b).start()

        def step(outer, carry):
            r0 = outer * _NBUF
            for b in range(_NBUF):
                r = r0 + b
                gdesc(r, b).wait()
                odesc(r, b).start()

                @pl.when(outer < n_outer - 1)
                def _(r=r, b=b):
                    odesc(r, b).wait()
                    gdesc(r + _NBUF, b).start()

            return carry

        lax.fori_loop(0, n_outer, step, 0)
        for b in range(_NBUF):
            odesc(rows_per_w - _NBUF + b, b).wait()

    return _emb


def kernel(input, weight):
    B, H = input.shape
    V, D = weight.shape
    NW = 32
    rows_per_w = B // NW
    eye2 = jnp.concatenate(
        [jnp.eye(D, dtype=weight.dtype), jnp.eye(D, dtype=weight.dtype)], axis=1)
    w128 = weight @ eye2
    out56 = _emb_call(B, H, D, rows_per_w)(input, w128)
    return out56[:, :H, :D]
